# baseline (device time: 50099 ns/iter reference)
import functools

import jax
import jax.numpy as jnp
from jax import lax
from jax.experimental import pallas as pl
from jax.experimental.pallas import tpu as pltpu

N_DEV = 4
B = 2
SQ = 128
H_LOC = 4
DH = 64
D_MODEL = 512
SKV_USED = 128
BLK = 64
SCALE = 0.125
NEG = -1e9


def kernel(x, Wq, K_ext, V_ext, Wo):
    kT = jnp.transpose(K_ext, (2, 0, 1, 3))
    vT = jnp.transpose(V_ext, (2, 0, 1, 3))

    def body(x_ref, wq_ref, k_ref, v_ref, wo_ref, out_ref,
             kv_buf, q_scr, ctx_scr, comm,
             send_sems, kv_recv_sems, ar_send_sems, ar_recv_sems):
        my = lax.axis_index("i")

        barrier_sem = pltpu.get_barrier_semaphore()
        for off in range(1, N_DEV):
            pl.semaphore_signal(
                barrier_sem, inc=1,
                device_id=((my + off) % N_DEV,),
                device_id_type=pl.DeviceIdType.MESH,
            )
        pl.semaphore_wait(barrier_sem, N_DEV - 1)

        def scatter_rdma(t, kv, src):
            return pltpu.make_async_remote_copy(
                src_ref=src.at[pl.ds(H_LOC * t, H_LOC)],
                dst_ref=kv_buf.at[kv],
                send_sem=send_sems.at[2 * (t - 1) + kv],
                recv_sem=kv_recv_sems.at[kv],
                device_id=(t,),
                device_id_type=pl.DeviceIdType.MESH,
            )

        @pl.when(my == 0)
        def _():
            for t in range(1, N_DEV):
                for kv, src in ((0, k_ref), (1, v_ref)):
                    scatter_rdma(t, kv, src).start()
            kv_buf[0] = k_ref[0:H_LOC]
            kv_buf[1] = v_ref[0:H_LOC]

        for b in range(B):
            q_scr[b] = lax.dot_general(
                x_ref[b], wq_ref[...],
                (((1,), (0,)), ((), ())),
                preferred_element_type=jnp.float32,
            )

        @pl.when(my != 0)
        def _():
            for kv in range(2):
                recv = pltpu.make_async_remote_copy(
                    src_ref=kv_buf.at[kv],
                    dst_ref=kv_buf.at[kv],
                    send_sem=send_sems.at[kv],
                    recv_sem=kv_recv_sems.at[kv],
                    device_id=(0,),
                    device_id_type=pl.DeviceIdType.MESH,
                )
                recv.wait_recv()

        rowb = lax.broadcasted_iota(jnp.int32, (SQ, SKV_USED), 0) // BLK
        colb = lax.broadcasted_iota(jnp.int32, (SQ, SKV_USED), 1) // BLK
        keep = colb <= rowb
        for b in range(B):
            for h in range(H_LOC):
                q_bh = q_scr[b, :, h * DH:(h + 1) * DH]
                k_bh = kv_buf[0, h, b]
                v_bh = kv_buf[1, h, b]
                s = lax.dot_general(
                    q_bh, k_bh, (((1,), (1,)), ((), ())),
                    preferred_element_type=jnp.float32,
                ) * SCALE
                s = jnp.where(keep, s, NEG)
                m = jnp.max(s, axis=1, keepdims=True)
                w = jnp.exp(s - m)
                w = w / jnp.sum(w, axis=1, keepdims=True)
                ctx_scr[b, :, h * DH:(h + 1) * DH] = lax.dot_general(
                    w, v_bh, (((1,), (0,)), ((), ())),
                    preferred_element_type=jnp.float32,
                )

        for b in range(B):
            out_ref[b] = lax.dot_general(
                ctx_scr[b], wo_ref[...],
                (((1,), (0,)), ((), ())),
                preferred_element_type=jnp.float32,
            )

        @pl.when(my == 0)
        def _():
            for t in range(1, N_DEV):
                for kv, src in ((0, k_ref), (1, v_ref)):
                    scatter_rdma(t, kv, src).wait_send()

        for stage, partner in enumerate((my ^ 1, 3 - my)):
            rdma = pltpu.make_async_remote_copy(
                src_ref=out_ref,
                dst_ref=comm.at[stage],
                send_sem=ar_send_sems.at[stage],
                recv_sem=ar_recv_sems.at[stage],
                device_id=(partner,),
                device_id_type=pl.DeviceIdType.MESH,
            )
            rdma.start()
            rdma.wait()
            out_ref[...] = out_ref[...] + comm[stage]

        @functools.partial(pl.run_scoped, sem=pltpu.SemaphoreType.REGULAR)
        def _(sem):
            for off in range(1, N_DEV):
                pl.semaphore_signal(
                    sem, inc=1,
                    device_id=((my + off) % N_DEV,),
                    device_id_type=pl.DeviceIdType.MESH,
                )
            pl.semaphore_wait(sem, N_DEV - 1)

    return pl.pallas_call(
        body,
        out_shape=jax.ShapeDtypeStruct((B, SQ, D_MODEL), jnp.float32),
        in_specs=[pl.BlockSpec(memory_space=pltpu.VMEM)] * 5,
        out_specs=pl.BlockSpec(memory_space=pltpu.VMEM),
        scratch_shapes=[
            pltpu.VMEM((2, H_LOC, B, SKV_USED, DH), jnp.float32),
            pltpu.VMEM((B, SQ, H_LOC * DH), jnp.float32),
            pltpu.VMEM((B, SQ, H_LOC * DH), jnp.float32),
            pltpu.VMEM((2, B, SQ, D_MODEL), jnp.float32),
            pltpu.SemaphoreType.DMA((2 * (N_DEV - 1),)),
            pltpu.SemaphoreType.DMA((2,)),
            pltpu.SemaphoreType.DMA((2,)),
            pltpu.SemaphoreType.DMA((2,)),
        ],
        compiler_params=pltpu.CompilerParams(collective_id=0),
    )(x, Wq, kT, vT, Wo)


# device time: 41828 ns/iter; 1.1977x vs baseline; 1.1977x over previous
import jax
import jax.numpy as jnp
from jax import lax
from jax.experimental import pallas as pl
from jax.experimental.pallas import tpu as pltpu

N_DEV = 4
B = 2
SQ = 128
H_LOC = 4
DH = 64
D_MODEL = 512
SKV_USED = 128
BLK = 64
SCALE = 0.125
NEG = -1e9


def kernel(x, Wq, K_ext, V_ext, Wo):
    kT = jnp.transpose(K_ext, (2, 0, 1, 3))
    vT = jnp.transpose(V_ext, (2, 0, 1, 3))

    def body(x_ref, wq_ref, k_ref, v_ref, wo_ref, out_ref,
             kv_buf, q_scr, ctx_scr, comm,
             send_sems, kv_recv_sems, local_sems, ar_send_sems, ar_recv_sems):
        my = lax.axis_index("i")

        barrier_sem = pltpu.get_barrier_semaphore()

        @pl.when(my != 0)
        def _():
            pl.semaphore_signal(
                barrier_sem, inc=1, device_id=(0,),
                device_id_type=pl.DeviceIdType.MESH,
            )

        @pl.when(my == 0)
        def _():
            pl.semaphore_wait(barrier_sem, N_DEV - 1)

        def scatter_rdma(t, kv, src):
            return pltpu.make_async_remote_copy(
                src_ref=src.at[pl.ds(H_LOC * t, H_LOC)],
                dst_ref=kv_buf.at[kv],
                send_sem=send_sems.at[2 * (t - 1) + kv],
                recv_sem=kv_recv_sems.at[kv],
                device_id=(t,),
                device_id_type=pl.DeviceIdType.MESH,
            )

        def local_kv_copy(kv, src):
            return pltpu.make_async_copy(
                src.at[pl.ds(0, H_LOC)], kv_buf.at[kv], local_sems.at[kv],
            )

        @pl.when(my == 0)
        def _():
            for t in (2, 1, 3):
                for kv, src in ((0, k_ref), (1, v_ref)):
                    scatter_rdma(t, kv, src).start()
            for kv, src in ((0, k_ref), (1, v_ref)):
                local_kv_copy(kv, src).start()

        for b in range(B):
            q_scr[b] = lax.dot_general(
                x_ref[b], wq_ref[...],
                (((1,), (0,)), ((), ())),
                preferred_element_type=jnp.float32,
            )

        @pl.when(my == 0)
        def _():
            for kv, src in ((0, k_ref), (1, v_ref)):
                local_kv_copy(kv, src).wait()

        @pl.when(my != 0)
        def _():
            for kv in range(2):
                recv = pltpu.make_async_remote_copy(
                    src_ref=kv_buf.at[kv],
                    dst_ref=kv_buf.at[kv],
                    send_sem=send_sems.at[kv],
                    recv_sem=kv_recv_sems.at[kv],
                    device_id=(0,),
                    device_id_type=pl.DeviceIdType.MESH,
                )
                recv.wait_recv()

        rowb = lax.broadcasted_iota(jnp.int32, (SQ, SKV_USED), 0) // BLK
        colb = lax.broadcasted_iota(jnp.int32, (SQ, SKV_USED), 1) // BLK
        keep = colb <= rowb
        for b in range(B):
            for h in range(H_LOC):
                q_bh = q_scr[b, :, h * DH:(h + 1) * DH]
                k_bh = kv_buf[0, h, b]
                v_bh = kv_buf[1, h, b]
                s = lax.dot_general(
                    q_bh, k_bh, (((1,), (1,)), ((), ())),
                    preferred_element_type=jnp.float32,
                ) * SCALE
                s = jnp.where(keep, s, NEG)
                m = jnp.max(s, axis=1, keepdims=True)
                w = jnp.exp(s - m)
                w = w / jnp.sum(w, axis=1, keepdims=True)
                ctx_scr[b, :, h * DH:(h + 1) * DH] = lax.dot_general(
                    w, v_bh, (((1,), (0,)), ((), ())),
                    preferred_element_type=jnp.float32,
                )

        @pl.when(my == 0)
        def _():
            for t in (2, 1, 3):
                for kv, src in ((0, k_ref), (1, v_ref)):
                    scatter_rdma(t, kv, src).wait_send()

        p_a = my ^ 1
        p_b = 3 - my

        def ar_rdma(stage, b, partner):
            return pltpu.make_async_remote_copy(
                src_ref=out_ref.at[b],
                dst_ref=comm.at[stage, b],
                send_sem=ar_send_sems.at[stage, b],
                recv_sem=ar_recv_sems.at[stage, b],
                device_id=(partner,),
                device_id_type=pl.DeviceIdType.MESH,
            )

        rdma_a = []
        for b in range(B):
            out_ref[b] = lax.dot_general(
                ctx_scr[b], wo_ref[...],
                (((1,), (0,)), ((), ())),
                preferred_element_type=jnp.float32,
            )
            r = ar_rdma(0, b, p_a)
            r.start()
            rdma_a.append(r)

        rdma_b = []
        for b in range(B):
            rdma_a[b].wait()
            out_ref[b] = out_ref[b] + comm[0, b]
            r = ar_rdma(1, b, p_b)
            r.start()
            rdma_b.append(r)

        for b in range(B):
            rdma_b[b].wait()
            out_ref[b] = out_ref[b] + comm[1, b]

    return pl.pallas_call(
        body,
        out_shape=jax.ShapeDtypeStruct((B, SQ, D_MODEL), jnp.float32),
        in_specs=[pl.BlockSpec(memory_space=pltpu.VMEM)] * 5,
        out_specs=pl.BlockSpec(memory_space=pltpu.VMEM),
        scratch_shapes=[
            pltpu.VMEM((2, H_LOC, B, SKV_USED, DH), jnp.float32),
            pltpu.VMEM((B, SQ, H_LOC * DH), jnp.float32),
            pltpu.VMEM((B, SQ, H_LOC * DH), jnp.float32),
            pltpu.VMEM((2, B, SQ, D_MODEL), jnp.float32),
            pltpu.SemaphoreType.DMA((2 * (N_DEV - 1),)),
            pltpu.SemaphoreType.DMA((2,)),
            pltpu.SemaphoreType.DMA((2,)),
            pltpu.SemaphoreType.DMA((2, B)),
            pltpu.SemaphoreType.DMA((2, B)),
        ],
        compiler_params=pltpu.CompilerParams(collective_id=0),
    )(x, Wq, kT, vT, Wo)


# device time: 40172 ns/iter; 1.2471x vs baseline; 1.0412x over previous
import jax
import jax.numpy as jnp
from jax import lax
from jax.experimental import pallas as pl
from jax.experimental.pallas import tpu as pltpu

N_DEV = 4
B = 2
SQ = 128
H_LOC = 4
DH = 64
D_MODEL = 512
BLK = 64
SCALE = 0.125


def kernel(x, Wq, K_ext, V_ext, Wo):
    def to_blocks(t):
        t = jnp.transpose(t, (2, 0, 1, 3))
        t = t.reshape(4 * H_LOC, B, 2, BLK, DH)
        return jnp.transpose(t, (2, 0, 1, 3, 4))

    kB = to_blocks(K_ext)
    vB = to_blocks(V_ext)

    def body(x_ref, wq_ref, k_ref, v_ref, wo_ref, out_ref,
             kv_buf, relay, q_scr, ctx_scr, comm,
             send_sems, kv_recv_sems, relay_recv_sems, fwd_send_sems,
             local_sems, ar_send_sems, ar_recv_sems):
        my = lax.axis_index("i")

        barrier_sem = pltpu.get_barrier_semaphore()

        @pl.when(my != 0)
        def _():
            pl.semaphore_signal(
                barrier_sem, inc=1, device_id=(0,),
                device_id_type=pl.DeviceIdType.MESH,
            )

        @pl.when(my == 0)
        def _():
            pl.semaphore_wait(barrier_sem, N_DEV - 1)

        def scatter_sends():
            src = {0: k_ref, 1: v_ref}
            plan = []
            for blk in range(2):
                plan.append((src[0].at[blk, pl.ds(8, H_LOC)],
                             relay.at[blk], relay_recv_sems.at[blk], 1))
                plan.append((src[1].at[blk, pl.ds(8, H_LOC)],
                             relay.at[blk], relay_recv_sems.at[blk], 3))
                for t in (1, 3):
                    for kv in range(2):
                        plan.append((src[kv].at[blk, pl.ds(H_LOC * t, H_LOC)],
                                     kv_buf.at[kv, blk],
                                     kv_recv_sems.at[kv, blk], t))
            return [
                pltpu.make_async_remote_copy(
                    src_ref=s, dst_ref=d, send_sem=send_sems.at[i],
                    recv_sem=r, device_id=(t,),
                    device_id_type=pl.DeviceIdType.MESH,
                )
                for i, (s, d, r, t) in enumerate(plan)
            ]

        def local_copy(kv, blk):
            src = {0: k_ref, 1: v_ref}[kv]
            return pltpu.make_async_copy(
                src.at[blk, pl.ds(0, H_LOC)],
                kv_buf.at[kv, blk],
                local_sems.at[kv, blk],
            )

        @pl.when(my == 0)
        def _():
            for r in scatter_sends():
                r.start()
            for kv in range(2):
                for blk in range(2):
                    local_copy(kv, blk).start()

        for b in range(B):
            q_scr[b] = lax.dot_general(
                x_ref[b], wq_ref[...],
                (((1,), (0,)), ((), ())),
                preferred_element_type=jnp.float32,
            )

        def fwd_rdma(kv, blk):
            return pltpu.make_async_remote_copy(
                src_ref=relay.at[blk],
                dst_ref=kv_buf.at[kv, blk],
                send_sem=fwd_send_sems.at[blk],
                recv_sem=kv_recv_sems.at[kv, blk],
                device_id=(2,),
                device_id_type=pl.DeviceIdType.MESH,
            )

        def relay_forward(blk):
            for dev, kv in ((1, 0), (3, 1)):
                @pl.when(my == dev)
                def _():
                    pltpu.make_async_remote_copy(
                        src_ref=relay.at[blk],
                        dst_ref=relay.at[blk],
                        send_sem=send_sems.at[0],
                        recv_sem=relay_recv_sems.at[blk],
                        device_id=(0,),
                        device_id_type=pl.DeviceIdType.MESH,
                    ).wait_recv()
                    fwd_rdma(kv, blk).start()

        def wait_kv(blk):
            @pl.when(my == 0)
            def _():
                for kv in range(2):
                    local_copy(kv, blk).wait()

            @pl.when(my != 0)
            def _():
                for kv in range(2):
                    pltpu.make_async_remote_copy(
                        src_ref=kv_buf.at[kv, blk],
                        dst_ref=kv_buf.at[kv, blk],
                        send_sem=send_sems.at[0],
                        recv_sem=kv_recv_sems.at[kv, blk],
                        device_id=(0,),
                        device_id_type=pl.DeviceIdType.MESH,
                    ).wait_recv()

        p_a = my ^ 1
        p_b = 3 - my

        def ar_rdma(stage, b, qb, partner):
            return pltpu.make_async_remote_copy(
                src_ref=out_ref.at[b, pl.ds(qb * BLK, BLK)],
                dst_ref=comm.at[stage, b, pl.ds(qb * BLK, BLK)],
                send_sem=ar_send_sems.at[stage, b, qb],
                recv_sem=ar_recv_sems.at[stage, b, qb],
                device_id=(partner,),
                device_id_type=pl.DeviceIdType.MESH,
            )

        def attention(b, h, qb):
            q = q_scr[b, qb * BLK:(qb + 1) * BLK, h * DH:(h + 1) * DH]
            k0 = kv_buf[0, 0, h, b]
            v0 = kv_buf[1, 0, h, b]
            s0 = lax.dot_general(
                q, k0, (((1,), (1,)), ((), ())),
                preferred_element_type=jnp.float32,
            ) * SCALE
            if qb == 0:
                m = jnp.max(s0, axis=1, keepdims=True)
                w = jnp.exp(s0 - m)
                ctx = lax.dot_general(
                    w, v0, (((1,), (0,)), ((), ())),
                    preferred_element_type=jnp.float32,
                ) / jnp.sum(w, axis=1, keepdims=True)
            else:
                k1 = kv_buf[0, 1, h, b]
                v1 = kv_buf[1, 1, h, b]
                s1 = lax.dot_general(
                    q, k1, (((1,), (1,)), ((), ())),
                    preferred_element_type=jnp.float32,
                ) * SCALE
                m = jnp.maximum(jnp.max(s0, axis=1, keepdims=True),
                                jnp.max(s1, axis=1, keepdims=True))
                w0 = jnp.exp(s0 - m)
                w1 = jnp.exp(s1 - m)
                num = lax.dot_general(
                    w0, v0, (((1,), (0,)), ((), ())),
                    preferred_element_type=jnp.float32,
                ) + lax.dot_general(
                    w1, v1, (((1,), (0,)), ((), ())),
                    preferred_element_type=jnp.float32,
                )
                ctx = num / (jnp.sum(w0, axis=1, keepdims=True)
                             + jnp.sum(w1, axis=1, keepdims=True))
            ctx_scr[b, qb * BLK:(qb + 1) * BLK, h * DH:(h + 1) * DH] = ctx

        rdma_a = {}

        def out_chunks(qb):
            for b in range(B):
                rows = pl.ds(qb * BLK, BLK)
                out_ref[b, rows] = lax.dot_general(
                    ctx_scr[b, qb * BLK:(qb + 1) * BLK, :], wo_ref[...],
                    (((1,), (0,)), ((), ())),
                    preferred_element_type=jnp.float32,
                )
                r = ar_rdma(0, b, qb, p_a)
                r.start()
                rdma_a[(b, qb)] = r

        relay_forward(0)
        wait_kv(0)
        for b in range(B):
            for h in range(H_LOC):
                attention(b, h, 0)
        relay_forward(1)
        out_chunks(0)
        wait_kv(1)
        for b in range(B):
            for h in range(H_LOC):
                attention(b, h, 1)
        out_chunks(1)

        @pl.when(my == 0)
        def _():
            for r in scatter_sends():
                r.wait_send()

        for dev, kv in ((1, 0), (3, 1)):
            @pl.when(my == dev)
            def _():
                for blk in range(2):
                    fwd_rdma(kv, blk).wait_send()

        chunks = [(0, 0), (1, 0), (0, 1), (1, 1)]
        rdma_b = {}
        for b, qb in chunks:
            rows = pl.ds(qb * BLK, BLK)
            rdma_a[(b, qb)].wait()
            out_ref[b, rows] = out_ref[b, rows] + comm[0, b, qb * BLK:(qb + 1) * BLK]
            r = ar_rdma(1, b, qb, p_b)
            r.start()
            rdma_b[(b, qb)] = r
        for b, qb in chunks:
            rows = pl.ds(qb * BLK, BLK)
            rdma_b[(b, qb)].wait()
            out_ref[b, rows] = out_ref[b, rows] + comm[1, b, qb * BLK:(qb + 1) * BLK]

    return pl.pallas_call(
        body,
        out_shape=jax.ShapeDtypeStruct((B, SQ, D_MODEL), jnp.float32),
        in_specs=[pl.BlockSpec(memory_space=pltpu.VMEM)] * 5,
        out_specs=pl.BlockSpec(memory_space=pltpu.VMEM),
        scratch_shapes=[
            pltpu.VMEM((2, 2, H_LOC, B, BLK, DH), jnp.float32),
            pltpu.VMEM((2, H_LOC, B, BLK, DH), jnp.float32),
            pltpu.VMEM((B, SQ, H_LOC * DH), jnp.float32),
            pltpu.VMEM((B, SQ, H_LOC * DH), jnp.float32),
            pltpu.VMEM((2, B, SQ, D_MODEL), jnp.float32),
            pltpu.SemaphoreType.DMA((12,)),
            pltpu.SemaphoreType.DMA((2, 2)),
            pltpu.SemaphoreType.DMA((2,)),
            pltpu.SemaphoreType.DMA((2,)),
            pltpu.SemaphoreType.DMA((2, 2)),
            pltpu.SemaphoreType.DMA((2, B, 2)),
            pltpu.SemaphoreType.DMA((2, B, 2)),
        ],
        compiler_params=pltpu.CompilerParams(collective_id=0),
    )(x, Wq, kB, vB, Wo)


# device time: 36714 ns/iter; 1.3646x vs baseline; 1.0942x over previous
import jax
import jax.numpy as jnp
from jax import lax
from jax.experimental import pallas as pl
from jax.experimental.pallas import tpu as pltpu

N_DEV = 4
B = 2
SQ = 128
H_LOC = 4
DH = 64
D_MODEL = 512
BLK = 64
SCALE = 0.125


def kernel(x, Wq, K_ext, V_ext, Wo):
    def body(x_ref, wq_ref, k_ref, v_ref, wo_ref, out_ref,
             x_v, wq_v, wo_v, kv_buf, relay, q_scr, ctx_scr, comm,
             in_sems, send_sems, kv_recv_sems, relay_recv_sems,
             fwd_send_sems, local_sems, ar_send_sems, ar_recv_sems):
        my = lax.axis_index("i")

        stage_in = [
            pltpu.make_async_copy(x_ref, x_v, in_sems.at[0]),
            pltpu.make_async_copy(wq_ref, wq_v, in_sems.at[1]),
            pltpu.make_async_copy(wo_ref, wo_v, in_sems.at[2]),
        ]
        for c in stage_in:
            c.start()

        barrier_sem = pltpu.get_barrier_semaphore()

        @pl.when(my != 0)
        def _():
            pl.semaphore_signal(
                barrier_sem, inc=1, device_id=(0,),
                device_id_type=pl.DeviceIdType.MESH,
            )

        @pl.when(my == 0)
        def _():
            pl.semaphore_wait(barrier_sem, N_DEV - 1)

        def src_slice(kv, blk, t):
            ref = {0: k_ref, 1: v_ref}[kv]
            return ref.at[:, pl.ds(blk * BLK, BLK), pl.ds(H_LOC * t, H_LOC), :]

        def dst_slice(kv, blk):
            return kv_buf.at[kv, :, pl.ds(blk * BLK, BLK)]

        def scatter_sends():
            plan = []
            for blk in range(2):
                plan.append((src_slice(0, blk, 2), relay.at[blk],
                             relay_recv_sems.at[blk], 1))
                plan.append((src_slice(1, blk, 2), relay.at[blk],
                             relay_recv_sems.at[blk], 3))
                for t in (1, 3):
                    for kv in range(2):
                        plan.append((src_slice(kv, blk, t), dst_slice(kv, blk),
                                     kv_recv_sems.at[kv, blk], t))
            return [
                pltpu.make_async_remote_copy(
                    src_ref=s, dst_ref=d, send_sem=send_sems.at[i],
                    recv_sem=r, device_id=(t,),
                    device_id_type=pl.DeviceIdType.MESH,
                )
                for i, (s, d, r, t) in enumerate(plan)
            ]

        def local_copy(kv, blk):
            return pltpu.make_async_copy(
                src_slice(kv, blk, 0), dst_slice(kv, blk),
                local_sems.at[kv, blk],
            )

        @pl.when(my == 0)
        def _():
            for r in scatter_sends():
                r.start()
            for kv in range(2):
                for blk in range(2):
                    local_copy(kv, blk).start()

        stage_in[0].wait()
        stage_in[1].wait()
        for b in range(B):
            q_scr[b] = lax.dot_general(
                x_v[b], wq_v[...],
                (((1,), (0,)), ((), ())),
                preferred_element_type=jnp.float32,
            )

        def fwd_rdma(kv, blk):
            return pltpu.make_async_remote_copy(
                src_ref=relay.at[blk],
                dst_ref=dst_slice(kv, blk),
                send_sem=fwd_send_sems.at[blk],
                recv_sem=kv_recv_sems.at[kv, blk],
                device_id=(2,),
                device_id_type=pl.DeviceIdType.MESH,
            )

        def relay_forward(blk):
            for dev, kv in ((1, 0), (3, 1)):
                @pl.when(my == dev)
                def _():
                    pltpu.make_async_remote_copy(
                        src_ref=relay.at[blk],
                        dst_ref=relay.at[blk],
                        send_sem=send_sems.at[0],
                        recv_sem=relay_recv_sems.at[blk],
                        device_id=(0,),
                        device_id_type=pl.DeviceIdType.MESH,
                    ).wait_recv()
                    fwd_rdma(kv, blk).start()

        def wait_kv(blk):
            @pl.when(my == 0)
            def _():
                for kv in range(2):
                    local_copy(kv, blk).wait()

            @pl.when(my != 0)
            def _():
                for kv in range(2):
                    pltpu.make_async_remote_copy(
                        src_ref=dst_slice(kv, blk),
                        dst_ref=dst_slice(kv, blk),
                        send_sem=send_sems.at[0],
                        recv_sem=kv_recv_sems.at[kv, blk],
                        device_id=(0,),
                        device_id_type=pl.DeviceIdType.MESH,
                    ).wait_recv()

        p_a = my ^ 1
        p_b = 3 - my

        def ar_rdma(stage, b, qb, partner):
            return pltpu.make_async_remote_copy(
                src_ref=out_ref.at[b, pl.ds(qb * BLK, BLK)],
                dst_ref=comm.at[stage, b, pl.ds(qb * BLK, BLK)],
                send_sem=ar_send_sems.at[stage, b, qb],
                recv_sem=ar_recv_sems.at[stage, b, qb],
                device_id=(partner,),
                device_id_type=pl.DeviceIdType.MESH,
            )

        def attention(b, h, qb):
            q = q_scr[b, qb * BLK:(qb + 1) * BLK, h * DH:(h + 1) * DH]
            k0 = kv_buf[0, b, 0:BLK, h, :]
            v0 = kv_buf[1, b, 0:BLK, h, :]
            s0 = lax.dot_general(
                q, k0, (((1,), (1,)), ((), ())),
                preferred_element_type=jnp.float32,
            ) * SCALE
            if qb == 0:
                m = jnp.max(s0, axis=1, keepdims=True)
                w = jnp.exp(s0 - m)
                ctx = lax.dot_general(
                    w, v0, (((1,), (0,)), ((), ())),
                    preferred_element_type=jnp.float32,
                ) / jnp.sum(w, axis=1, keepdims=True)
            else:
                k1 = kv_buf[0, b, BLK:SQ, h, :]
                v1 = kv_buf[1, b, BLK:SQ, h, :]
                s1 = lax.dot_general(
                    q, k1, (((1,), (1,)), ((), ())),
                    preferred_element_type=jnp.float32,
                ) * SCALE
                m = jnp.maximum(jnp.max(s0, axis=1, keepdims=True),
                                jnp.max(s1, axis=1, keepdims=True))
                w0 = jnp.exp(s0 - m)
                w1 = jnp.exp(s1 - m)
                num = lax.dot_general(
                    w0, v0, (((1,), (0,)), ((), ())),
                    preferred_element_type=jnp.float32,
                ) + lax.dot_general(
                    w1, v1, (((1,), (0,)), ((), ())),
                    preferred_element_type=jnp.float32,
                )
                ctx = num / (jnp.sum(w0, axis=1, keepdims=True)
                             + jnp.sum(w1, axis=1, keepdims=True))
            ctx_scr[b, qb * BLK:(qb + 1) * BLK, h * DH:(h + 1) * DH] = ctx

        rdma_a = {}

        def out_chunks(qb):
            for b in range(B):
                rows = pl.ds(qb * BLK, BLK)
                out_ref[b, rows] = lax.dot_general(
                    ctx_scr[b, qb * BLK:(qb + 1) * BLK, :], wo_v[...],
                    (((1,), (0,)), ((), ())),
                    preferred_element_type=jnp.float32,
                )
                r = ar_rdma(0, b, qb, p_a)
                r.start()
                rdma_a[(b, qb)] = r

        relay_forward(0)
        wait_kv(0)
        for b in range(B):
            for h in range(H_LOC):
                attention(b, h, 0)
        relay_forward(1)
        stage_in[2].wait()
        out_chunks(0)
        wait_kv(1)
        for b in range(B):
            for h in range(H_LOC):
                attention(b, h, 1)
        out_chunks(1)

        @pl.when(my == 0)
        def _():
            for r in scatter_sends():
                r.wait_send()

        for dev, kv in ((1, 0), (3, 1)):
            @pl.when(my == dev)
            def _():
                for blk in range(2):
                    fwd_rdma(kv, blk).wait_send()

        chunks = [(0, 0), (1, 0), (0, 1), (1, 1)]
        rdma_b = {}
        for b, qb in chunks:
            rows = pl.ds(qb * BLK, BLK)
            rdma_a[(b, qb)].wait()
            out_ref[b, rows] = out_ref[b, rows] + comm[0, b, qb * BLK:(qb + 1) * BLK]
            r = ar_rdma(1, b, qb, p_b)
            r.start()
            rdma_b[(b, qb)] = r
        for b, qb in chunks:
            rows = pl.ds(qb * BLK, BLK)
            rdma_b[(b, qb)].wait()
            out_ref[b, rows] = out_ref[b, rows] + comm[1, b, qb * BLK:(qb + 1) * BLK]

    return pl.pallas_call(
        body,
        out_shape=jax.ShapeDtypeStruct((B, SQ, D_MODEL), jnp.float32),
        in_specs=[pl.BlockSpec(memory_space=pl.ANY)] * 5,
        out_specs=pl.BlockSpec(memory_space=pltpu.VMEM),
        scratch_shapes=[
            pltpu.VMEM((B, SQ, D_MODEL), jnp.float32),
            pltpu.VMEM((D_MODEL, H_LOC * DH), jnp.float32),
            pltpu.VMEM((H_LOC * DH, D_MODEL), jnp.float32),
            pltpu.VMEM((2, B, SQ, H_LOC, DH), jnp.float32),
            pltpu.VMEM((2, B, BLK, H_LOC, DH), jnp.float32),
            pltpu.VMEM((B, SQ, H_LOC * DH), jnp.float32),
            pltpu.VMEM((B, SQ, H_LOC * DH), jnp.float32),
            pltpu.VMEM((2, B, SQ, D_MODEL), jnp.float32),
            pltpu.SemaphoreType.DMA((3,)),
            pltpu.SemaphoreType.DMA((12,)),
            pltpu.SemaphoreType.DMA((2, 2)),
            pltpu.SemaphoreType.DMA((2,)),
            pltpu.SemaphoreType.DMA((2,)),
            pltpu.SemaphoreType.DMA((2, 2)),
            pltpu.SemaphoreType.DMA((2, B, 2)),
            pltpu.SemaphoreType.DMA((2, B, 2)),
        ],
        compiler_params=pltpu.CompilerParams(collective_id=0),
    )(x, Wq, K_ext, V_ext, Wo)


# device time: 33369 ns/iter; 1.5014x vs baseline; 1.1002x over previous
import jax
import jax.numpy as jnp
from jax import lax
from jax.experimental import pallas as pl
from jax.experimental.pallas import tpu as pltpu

N_DEV = 4
B = 2
SQ = 128
H_LOC = 4
DH = 64
D_MODEL = 512
BLK = 64
SCALE = 0.125


def kernel(x, Wq, K_ext, V_ext, Wo):
    kT = lax.transpose(K_ext, (0, 2, 3, 1))
    vT = lax.transpose(V_ext, (0, 2, 3, 1))

    def body(x_ref, wq_ref, k_ref, v_ref, wo_ref, out_ref,
             x_v, wq_v, wo_v, kv_buf, q_scr, ctx_scr, acc, comm,
             in_sems, out_sems, send_sems, kv_recv_sems, relay_recv_sems,
             fwd_send_sems, local_sems, ar_send_sems, ar_recv_sems):
        my = lax.axis_index("i")

        stage_in = [
            pltpu.make_async_copy(x_ref, x_v, in_sems.at[0]),
            pltpu.make_async_copy(wq_ref, wq_v, in_sems.at[1]),
            pltpu.make_async_copy(wo_ref, wo_v, in_sems.at[2]),
        ]
        for c in stage_in:
            c.start()

        barrier_sem = pltpu.get_barrier_semaphore()

        @pl.when(my != 0)
        def _():
            pl.semaphore_signal(
                barrier_sem, inc=1, device_id=(0,),
                device_id_type=pl.DeviceIdType.MESH,
            )

        @pl.when(my == 0)
        def _():
            pl.semaphore_wait(barrier_sem, N_DEV - 1)

        def src_slice(kv, c, t):
            ref = {0: k_ref, 1: v_ref}[kv]
            return ref.at[:, pl.ds(H_LOC * t + 2 * c, 2), :, :]

        def dst_slice(kv, c):
            return kv_buf.at[kv, c]

        def scatter_sends():
            plan = []
            for c in range(2):
                plan.append((src_slice(0, c, 2), kv_buf.at[2, c],
                             relay_recv_sems.at[c], 1))
                plan.append((src_slice(1, c, 2), kv_buf.at[2, c],
                             relay_recv_sems.at[c], 3))
                for t in (1, 3):
                    for kv in range(2):
                        plan.append((src_slice(kv, c, t), dst_slice(kv, c),
                                     kv_recv_sems.at[kv, c], t))
            return [
                pltpu.make_async_remote_copy(
                    src_ref=s, dst_ref=d, send_sem=send_sems.at[i],
                    recv_sem=r, device_id=(t,),
                    device_id_type=pl.DeviceIdType.MESH,
                )
                for i, (s, d, r, t) in enumerate(plan)
            ]

        def local_copy(kv, c):
            return pltpu.make_async_copy(
                src_slice(kv, c, 0), dst_slice(kv, c),
                local_sems.at[kv, c],
            )

        @pl.when(my == 0)
        def _():
            for r in scatter_sends():
                r.start()
            for kv in range(2):
                for c in range(2):
                    local_copy(kv, c).start()

        stage_in[0].wait()
        stage_in[1].wait()
        for b in range(B):
            q_scr[b] = lax.dot_general(
                x_v[b], wq_v[...],
                (((1,), (0,)), ((), ())),
                preferred_element_type=jnp.float32,
            )

        def fwd_rdma(kv, c):
            return pltpu.make_async_remote_copy(
                src_ref=kv_buf.at[2, c],
                dst_ref=dst_slice(kv, c),
                send_sem=fwd_send_sems.at[c],
                recv_sem=kv_recv_sems.at[kv, c],
                device_id=(2,),
                device_id_type=pl.DeviceIdType.MESH,
            )

        def relay_forward(c):
            for dev, kv in ((1, 0), (3, 1)):
                @pl.when(my == dev)
                def _():
                    pltpu.make_async_remote_copy(
                        src_ref=kv_buf.at[2, c],
                        dst_ref=kv_buf.at[2, c],
                        send_sem=send_sems.at[0],
                        recv_sem=relay_recv_sems.at[c],
                        device_id=(0,),
                        device_id_type=pl.DeviceIdType.MESH,
                    ).wait_recv()
                    fwd_rdma(kv, c).start()

        def wait_kv(c):
            @pl.when(my == 0)
            def _():
                for kv in range(2):
                    local_copy(kv, c).wait()

            @pl.when(my != 0)
            def _():
                for kv in range(2):
                    pltpu.make_async_remote_copy(
                        src_ref=dst_slice(kv, c),
                        dst_ref=dst_slice(kv, c),
                        send_sem=send_sems.at[0],
                        recv_sem=kv_recv_sems.at[kv, c],
                        device_id=(0,),
                        device_id_type=pl.DeviceIdType.MESH,
                    ).wait_recv()

        p_a = my ^ 1
        p_b = 3 - my

        def ar_rdma(stage, b, qb, partner):
            return pltpu.make_async_remote_copy(
                src_ref=acc.at[b, pl.ds(qb * BLK, BLK)],
                dst_ref=comm.at[stage, b, pl.ds(qb * BLK, BLK)],
                send_sem=ar_send_sems.at[stage, b, qb],
                recv_sem=ar_recv_sems.at[stage, b, qb],
                device_id=(partner,),
                device_id_type=pl.DeviceIdType.MESH,
            )

        def attention(b, h, qb):
            c, sub = h // 2, h % 2
            q = q_scr[b, qb * BLK:(qb + 1) * BLK, h * DH:(h + 1) * DH]
            k0 = kv_buf[0, c, b, sub, :, 0:BLK]
            v0 = kv_buf[1, c, b, sub, :, 0:BLK]
            s0 = lax.dot_general(
                q, k0, (((1,), (0,)), ((), ())),
                preferred_element_type=jnp.float32,
            ) * SCALE
            if qb == 0:
                m = jnp.max(s0, axis=1, keepdims=True)
                w = jnp.exp(s0 - m)
                ctx = lax.dot_general(
                    w, v0, (((1,), (1,)), ((), ())),
                    preferred_element_type=jnp.float32,
                ) / jnp.sum(w, axis=1, keepdims=True)
            else:
                k1 = kv_buf[0, c, b, sub, :, BLK:SQ]
                v1 = kv_buf[1, c, b, sub, :, BLK:SQ]
                s1 = lax.dot_general(
                    q, k1, (((1,), (0,)), ((), ())),
                    preferred_element_type=jnp.float32,
                ) * SCALE
                m = jnp.maximum(jnp.max(s0, axis=1, keepdims=True),
                                jnp.max(s1, axis=1, keepdims=True))
                w0 = jnp.exp(s0 - m)
                w1 = jnp.exp(s1 - m)
                num = lax.dot_general(
                    w0, v0, (((1,), (1,)), ((), ())),
                    preferred_element_type=jnp.float32,
                ) + lax.dot_general(
                    w1, v1, (((1,), (1,)), ((), ())),
                    preferred_element_type=jnp.float32,
                )
                ctx = num / (jnp.sum(w0, axis=1, keepdims=True)
                             + jnp.sum(w1, axis=1, keepdims=True))
            ctx_scr[b, qb * BLK:(qb + 1) * BLK, h * DH:(h + 1) * DH] = ctx

        rdma_a = {}

        def out_chunks(qb):
            for b in range(B):
                rows = pl.ds(qb * BLK, BLK)
                acc[b, rows] = lax.dot_general(
                    ctx_scr[b, qb * BLK:(qb + 1) * BLK, :], wo_v[...],
                    (((1,), (0,)), ((), ())),
                    preferred_element_type=jnp.float32,
                )
                r = ar_rdma(0, b, qb, p_a)
                r.start()
                rdma_a[(b, qb)] = r

        relay_forward(0)
        wait_kv(0)
        for b in range(B):
            for h in (0, 1):
                attention(b, h, 0)
                attention(b, h, 1)
        relay_forward(1)
        wait_kv(1)
        for b in range(B):
            for h in (2, 3):
                attention(b, h, 0)
                attention(b, h, 1)
        stage_in[2].wait()
        out_chunks(0)
        out_chunks(1)

        @pl.when(my == 0)
        def _():
            for r in scatter_sends():
                r.wait_send()

        for dev, kv in ((1, 0), (3, 1)):
            @pl.when(my == dev)
            def _():
                for c in range(2):
                    fwd_rdma(kv, c).wait_send()

        chunks = [(0, 0), (1, 0), (0, 1), (1, 1)]
        rdma_b = {}
        for b, qb in chunks:
            rows = pl.ds(qb * BLK, BLK)
            rdma_a[(b, qb)].wait()
            acc[b, rows] = acc[b, rows] + comm[0, b, qb * BLK:(qb + 1) * BLK]
            r = ar_rdma(1, b, qb, p_b)
            r.start()
            rdma_b[(b, qb)] = r
        out_copies = []
        for b, qb in chunks:
            rows = pl.ds(qb * BLK, BLK)
            rdma_b[(b, qb)].wait()
            acc[b, rows] = acc[b, rows] + comm[1, b, qb * BLK:(qb + 1) * BLK]
            c = pltpu.make_async_copy(
                acc.at[b, rows], out_ref.at[b, rows], out_sems.at[b, qb],
            )
            c.start()
            out_copies.append(c)
        for c in out_copies:
            c.wait()

    return pl.pallas_call(
        body,
        out_shape=jax.ShapeDtypeStruct((B, SQ, D_MODEL), jnp.float32),
        in_specs=[pl.BlockSpec(memory_space=pl.ANY)] * 5,
        out_specs=pl.BlockSpec(memory_space=pl.ANY),
        scratch_shapes=[
            pltpu.VMEM((B, SQ, D_MODEL), jnp.float32),
            pltpu.VMEM((D_MODEL, H_LOC * DH), jnp.float32),
            pltpu.VMEM((H_LOC * DH, D_MODEL), jnp.float32),
            pltpu.VMEM((3, 2, B, 2, DH, SQ), jnp.float32),
            pltpu.VMEM((B, SQ, H_LOC * DH), jnp.float32),
            pltpu.VMEM((B, SQ, H_LOC * DH), jnp.float32),
            pltpu.VMEM((B, SQ, D_MODEL), jnp.float32),
            pltpu.VMEM((2, B, SQ, D_MODEL), jnp.float32),
            pltpu.SemaphoreType.DMA((3,)),
            pltpu.SemaphoreType.DMA((B, 2)),
            pltpu.SemaphoreType.DMA((12,)),
            pltpu.SemaphoreType.DMA((2, 2)),
            pltpu.SemaphoreType.DMA((2,)),
            pltpu.SemaphoreType.DMA((2,)),
            pltpu.SemaphoreType.DMA((2, 2)),
            pltpu.SemaphoreType.DMA((2, B, 2)),
            pltpu.SemaphoreType.DMA((2, B, 2)),
        ],
        compiler_params=pltpu.CompilerParams(collective_id=0),
    )(x, Wq, kT, vT, Wo)


# device time: 33175 ns/iter; 1.5101x vs baseline; 1.0058x over previous
import jax
import jax.numpy as jnp
from jax import lax
from jax.experimental import pallas as pl
from jax.experimental.pallas import tpu as pltpu

N_DEV = 4
B = 2
SQ = 128
H_LOC = 4
DH = 64
D_MODEL = 512
BLK = 64
SCALE = 0.125


def kernel(x, Wq, K_ext, V_ext, Wo):
    kT = lax.transpose(K_ext, (0, 2, 3, 1))
    vT = lax.transpose(V_ext, (0, 2, 3, 1))

    def body(x_ref, wq_ref, k_ref, v_ref, wo_ref, out_ref,
             x_v, wq_v, wo_v, kv_buf, q_scr, ctx_scr, comm,
             in_sems, send_sems, kv_recv_sems, relay_recv_sems,
             fwd_send_sems, local_sems, ar_send_sems, ar_recv_sems):
        my = lax.axis_index("i")

        stage_in = [
            pltpu.make_async_copy(x_ref, x_v, in_sems.at[0]),
            pltpu.make_async_copy(wq_ref, wq_v, in_sems.at[1]),
            pltpu.make_async_copy(wo_ref, wo_v, in_sems.at[2]),
        ]
        for c in stage_in:
            c.start()

        barrier_sem = pltpu.get_barrier_semaphore()

        @pl.when(my != 0)
        def _():
            pl.semaphore_signal(
                barrier_sem, inc=1, device_id=(0,),
                device_id_type=pl.DeviceIdType.MESH,
            )

        @pl.when(my == 0)
        def _():
            pl.semaphore_wait(barrier_sem, N_DEV - 1)

        def src_slice(kv, c, t):
            ref = {0: k_ref, 1: v_ref}[kv]
            return ref.at[:, pl.ds(H_LOC * t + 2 * c, 2), :, :]

        def dst_slice(kv, c):
            return kv_buf.at[kv, c]

        def scatter_sends():
            plan = []
            for c in range(2):
                plan.append((src_slice(0, c, 2), kv_buf.at[2, c],
                             relay_recv_sems.at[c], 1))
                plan.append((src_slice(1, c, 2), kv_buf.at[2, c],
                             relay_recv_sems.at[c], 3))
                for t in (1, 3):
                    for kv in range(2):
                        plan.append((src_slice(kv, c, t), dst_slice(kv, c),
                                     kv_recv_sems.at[kv, c], t))
            return [
                pltpu.make_async_remote_copy(
                    src_ref=s, dst_ref=d, send_sem=send_sems.at[i],
                    recv_sem=r, device_id=(t,),
                    device_id_type=pl.DeviceIdType.MESH,
                )
                for i, (s, d, r, t) in enumerate(plan)
            ]

        def local_copy(kv, c):
            return pltpu.make_async_copy(
                src_slice(kv, c, 0), dst_slice(kv, c),
                local_sems.at[kv, c],
            )

        @pl.when(my == 0)
        def _():
            for r in scatter_sends():
                r.start()
            for kv in range(2):
                for c in range(2):
                    local_copy(kv, c).start()

        stage_in[0].wait()
        stage_in[1].wait()
        for b in range(B):
            q_scr[b] = lax.dot_general(
                x_v[b], wq_v[...],
                (((1,), (0,)), ((), ())),
                preferred_element_type=jnp.float32,
            )

        def fwd_rdma(kv, c):
            return pltpu.make_async_remote_copy(
                src_ref=kv_buf.at[2, c],
                dst_ref=dst_slice(kv, c),
                send_sem=fwd_send_sems.at[c],
                recv_sem=kv_recv_sems.at[kv, c],
                device_id=(2,),
                device_id_type=pl.DeviceIdType.MESH,
            )

        def relay_forward(c):
            for dev, kv in ((1, 0), (3, 1)):
                @pl.when(my == dev)
                def _():
                    pltpu.make_async_remote_copy(
                        src_ref=kv_buf.at[2, c],
                        dst_ref=kv_buf.at[2, c],
                        send_sem=send_sems.at[0],
                        recv_sem=relay_recv_sems.at[c],
                        device_id=(0,),
                        device_id_type=pl.DeviceIdType.MESH,
                    ).wait_recv()
                    fwd_rdma(kv, c).start()

        def wait_kv(c):
            @pl.when(my == 0)
            def _():
                for kv in range(2):
                    local_copy(kv, c).wait()

            @pl.when(my != 0)
            def _():
                for kv in range(2):
                    pltpu.make_async_remote_copy(
                        src_ref=dst_slice(kv, c),
                        dst_ref=dst_slice(kv, c),
                        send_sem=send_sems.at[0],
                        recv_sem=kv_recv_sems.at[kv, c],
                        device_id=(0,),
                        device_id_type=pl.DeviceIdType.MESH,
                    ).wait_recv()

        p_a = my ^ 1
        p_b = 3 - my

        def ar_rdma(stage, b, qb, partner):
            return pltpu.make_async_remote_copy(
                src_ref=out_ref.at[b, pl.ds(qb * BLK, BLK)],
                dst_ref=comm.at[stage, b, pl.ds(qb * BLK, BLK)],
                send_sem=ar_send_sems.at[stage, b, qb],
                recv_sem=ar_recv_sems.at[stage, b, qb],
                device_id=(partner,),
                device_id_type=pl.DeviceIdType.MESH,
            )

        def attention(b, h, qb):
            c, sub = h // 2, h % 2
            q = q_scr[b, qb * BLK:(qb + 1) * BLK, h * DH:(h + 1) * DH]
            k0 = kv_buf[0, c, b, sub, :, 0:BLK]
            v0 = kv_buf[1, c, b, sub, :, 0:BLK]
            s0 = lax.dot_general(
                q, k0, (((1,), (0,)), ((), ())),
                preferred_element_type=jnp.float32,
            ) * SCALE
            if qb == 0:
                m = jnp.max(s0, axis=1, keepdims=True)
                w = jnp.exp(s0 - m)
                ctx = lax.dot_general(
                    w, v0, (((1,), (1,)), ((), ())),
                    preferred_element_type=jnp.float32,
                ) / jnp.sum(w, axis=1, keepdims=True)
            else:
                k1 = kv_buf[0, c, b, sub, :, BLK:SQ]
                v1 = kv_buf[1, c, b, sub, :, BLK:SQ]
                s1 = lax.dot_general(
                    q, k1, (((1,), (0,)), ((), ())),
                    preferred_element_type=jnp.float32,
                ) * SCALE
                m = jnp.maximum(jnp.max(s0, axis=1, keepdims=True),
                                jnp.max(s1, axis=1, keepdims=True))
                w0 = jnp.exp(s0 - m)
                w1 = jnp.exp(s1 - m)
                num = lax.dot_general(
                    w0, v0, (((1,), (1,)), ((), ())),
                    preferred_element_type=jnp.float32,
                ) + lax.dot_general(
                    w1, v1, (((1,), (1,)), ((), ())),
                    preferred_element_type=jnp.float32,
                )
                ctx = num / (jnp.sum(w0, axis=1, keepdims=True)
                             + jnp.sum(w1, axis=1, keepdims=True))
            ctx_scr[b, qb * BLK:(qb + 1) * BLK, h * DH:(h + 1) * DH] = ctx

        rdma_a = {}

        def out_chunks(qb):
            for b in range(B):
                rows = pl.ds(qb * BLK, BLK)
                out_ref[b, rows] = lax.dot_general(
                    ctx_scr[b, qb * BLK:(qb + 1) * BLK, :], wo_v[...],
                    (((1,), (0,)), ((), ())),
                    preferred_element_type=jnp.float32,
                )
                r = ar_rdma(0, b, qb, p_a)
                r.start()
                rdma_a[(b, qb)] = r

        relay_forward(0)
        wait_kv(0)
        for b in range(B):
            for h in (0, 1):
                attention(b, h, 0)
                attention(b, h, 1)
        relay_forward(1)
        wait_kv(1)
        for b in range(B):
            for h in (2, 3):
                attention(b, h, 0)
                attention(b, h, 1)
        stage_in[2].wait()
        out_chunks(0)
        out_chunks(1)

        @pl.when(my == 0)
        def _():
            for r in scatter_sends():
                r.wait_send()

        for dev, kv in ((1, 0), (3, 1)):
            @pl.when(my == dev)
            def _():
                for c in range(2):
                    fwd_rdma(kv, c).wait_send()

        chunks = [(0, 0), (1, 0), (0, 1), (1, 1)]
        rdma_b = {}
        for b, qb in chunks:
            rows = pl.ds(qb * BLK, BLK)
            rdma_a[(b, qb)].wait()
            out_ref[b, rows] = out_ref[b, rows] + comm[0, b, qb * BLK:(qb + 1) * BLK]
            r = ar_rdma(1, b, qb, p_b)
            r.start()
            rdma_b[(b, qb)] = r
        for b, qb in chunks:
            rows = pl.ds(qb * BLK, BLK)
            rdma_b[(b, qb)].wait()
            out_ref[b, rows] = out_ref[b, rows] + comm[1, b, qb * BLK:(qb + 1) * BLK]

    return pl.pallas_call(
        body,
        out_shape=jax.ShapeDtypeStruct((B, SQ, D_MODEL), jnp.float32),
        in_specs=[pl.BlockSpec(memory_space=pl.ANY)] * 5,
        out_specs=pl.BlockSpec(memory_space=pltpu.VMEM),
        scratch_shapes=[
            pltpu.VMEM((B, SQ, D_MODEL), jnp.float32),
            pltpu.VMEM((D_MODEL, H_LOC * DH), jnp.float32),
            pltpu.VMEM((H_LOC * DH, D_MODEL), jnp.float32),
            pltpu.VMEM((3, 2, B, 2, DH, SQ), jnp.float32),
            pltpu.VMEM((B, SQ, H_LOC * DH), jnp.float32),
            pltpu.VMEM((B, SQ, H_LOC * DH), jnp.float32),
            pltpu.VMEM((2, B, SQ, D_MODEL), jnp.float32),
            pltpu.SemaphoreType.DMA((3,)),
            pltpu.SemaphoreType.DMA((12,)),
            pltpu.SemaphoreType.DMA((2, 2)),
            pltpu.SemaphoreType.DMA((2,)),
            pltpu.SemaphoreType.DMA((2,)),
            pltpu.SemaphoreType.DMA((2, 2)),
            pltpu.SemaphoreType.DMA((2, B, 2)),
            pltpu.SemaphoreType.DMA((2, B, 2)),
        ],
        compiler_params=pltpu.CompilerParams(collective_id=0),
    )(x, Wq, kT, vT, Wo)


# device time: 29628 ns/iter; 1.6909x vs baseline; 1.1197x over previous
import jax
import jax.numpy as jnp
from jax import lax
from jax.experimental import pallas as pl
from jax.experimental.pallas import tpu as pltpu

N_DEV = 4
B = 2
SQ = 128
H_LOC = 4
DH = 64
D_MODEL = 512
BLK = 64
SCALE = 0.125


def kernel(x, Wq, K_ext, V_ext, Wo):
    kT = lax.transpose(K_ext, (0, 2, 3, 1))
    vT = lax.transpose(V_ext, (0, 2, 3, 1))

    def body(x_ref, wq_ref, k_ref, v_ref, wo_ref, out_ref,
             x_v, wq_v, wo_v, kv_buf, q_scr, ctx_scr, comm, sendbuf,
             in_sems, send_sems, kv_recv_sems, relay_recv_sems,
             fwd_send_sems, local_sems, ar_send_sems, ar_recv_sems):
        my = lax.axis_index("i")

        stage_in = [
            pltpu.make_async_copy(x_ref, x_v, in_sems.at[0]),
            pltpu.make_async_copy(wq_ref, wq_v, in_sems.at[1]),
            pltpu.make_async_copy(wo_ref, wo_v, in_sems.at[2]),
        ]
        for c in stage_in:
            c.start()

        barrier_sem = pltpu.get_barrier_semaphore()

        @pl.when(my != 0)
        def _():
            pl.semaphore_signal(
                barrier_sem, inc=1, device_id=(0,),
                device_id_type=pl.DeviceIdType.MESH,
            )

        @pl.when(my == 0)
        def _():
            pl.semaphore_wait(barrier_sem, N_DEV - 1)

        def src_slice(kv, c, t):
            ref = {0: k_ref, 1: v_ref}[kv]
            return ref.at[:, pl.ds(H_LOC * t + 2 * c, 2), :, :]

        def dst_slice(kv, c):
            return kv_buf.at[kv, c]

        def scatter_sends():
            plan = []
            for c in range(2):
                plan.append((src_slice(0, c, 2), kv_buf.at[2, c],
                             relay_recv_sems.at[c], 1))
                plan.append((src_slice(1, c, 2), kv_buf.at[2, c],
                             relay_recv_sems.at[c], 3))
                for t in (1, 3):
                    for kv in range(2):
                        plan.append((src_slice(kv, c, t), dst_slice(kv, c),
                                     kv_recv_sems.at[kv, c], t))
            return [
                pltpu.make_async_remote_copy(
                    src_ref=s, dst_ref=d, send_sem=send_sems.at[i],
                    recv_sem=r, device_id=(t,),
                    device_id_type=pl.DeviceIdType.MESH,
                )
                for i, (s, d, r, t) in enumerate(plan)
            ]

        def local_copy(kv, c):
            return pltpu.make_async_copy(
                src_slice(kv, c, 0), dst_slice(kv, c),
                local_sems.at[kv, c],
            )

        @pl.when(my == 0)
        def _():
            for r in scatter_sends():
                r.start()
            for kv in range(2):
                for c in range(2):
                    local_copy(kv, c).start()

        stage_in[0].wait()
        stage_in[1].wait()
        for b in range(B):
            q_scr[b] = lax.dot_general(
                x_v[b], wq_v[...],
                (((1,), (0,)), ((), ())),
                preferred_element_type=jnp.float32,
            )

        def fwd_rdma(kv, c):
            return pltpu.make_async_remote_copy(
                src_ref=kv_buf.at[2, c],
                dst_ref=dst_slice(kv, c),
                send_sem=fwd_send_sems.at[c],
                recv_sem=kv_recv_sems.at[kv, c],
                device_id=(2,),
                device_id_type=pl.DeviceIdType.MESH,
            )

        def relay_forward(c):
            for dev, kv in ((1, 0), (3, 1)):
                @pl.when(my == dev)
                def _():
                    pltpu.make_async_remote_copy(
                        src_ref=kv_buf.at[2, c],
                        dst_ref=kv_buf.at[2, c],
                        send_sem=send_sems.at[0],
                        recv_sem=relay_recv_sems.at[c],
                        device_id=(0,),
                        device_id_type=pl.DeviceIdType.MESH,
                    ).wait_recv()
                    fwd_rdma(kv, c).start()

        def wait_kv(c):
            @pl.when(my == 0)
            def _():
                for kv in range(2):
                    local_copy(kv, c).wait()

            @pl.when(my != 0)
            def _():
                for kv in range(2):
                    pltpu.make_async_remote_copy(
                        src_ref=dst_slice(kv, c),
                        dst_ref=dst_slice(kv, c),
                        send_sem=send_sems.at[0],
                        recv_sem=kv_recv_sems.at[kv, c],
                        device_id=(0,),
                        device_id_type=pl.DeviceIdType.MESH,
                    ).wait_recv()

        p_a = my ^ 1
        p_b = 3 - my

        def ar_rdma(stage, b, qb, partner):
            return pltpu.make_async_remote_copy(
                src_ref=sendbuf.at[stage, b, pl.ds(qb * BLK, BLK)],
                dst_ref=comm.at[stage, b, pl.ds(qb * BLK, BLK)],
                send_sem=ar_send_sems.at[stage, b, qb],
                recv_sem=ar_recv_sems.at[stage, b, qb],
                device_id=(partner,),
                device_id_type=pl.DeviceIdType.MESH,
            )

        def attention(b, h, qb):
            c, sub = h // 2, h % 2
            q = q_scr[b, qb * BLK:(qb + 1) * BLK, h * DH:(h + 1) * DH]
            k0 = kv_buf[0, c, b, sub, :, 0:BLK]
            v0 = kv_buf[1, c, b, sub, :, 0:BLK]
            s0 = lax.dot_general(
                q, k0, (((1,), (0,)), ((), ())),
                preferred_element_type=jnp.float32,
            ) * SCALE
            if qb == 0:
                m = jnp.max(s0, axis=1, keepdims=True)
                w = jnp.exp(s0 - m)
                ctx = lax.dot_general(
                    w, v0, (((1,), (1,)), ((), ())),
                    preferred_element_type=jnp.float32,
                ) / jnp.sum(w, axis=1, keepdims=True)
            else:
                k1 = kv_buf[0, c, b, sub, :, BLK:SQ]
                v1 = kv_buf[1, c, b, sub, :, BLK:SQ]
                s1 = lax.dot_general(
                    q, k1, (((1,), (0,)), ((), ())),
                    preferred_element_type=jnp.float32,
                ) * SCALE
                m = jnp.maximum(jnp.max(s0, axis=1, keepdims=True),
                                jnp.max(s1, axis=1, keepdims=True))
                w0 = jnp.exp(s0 - m)
                w1 = jnp.exp(s1 - m)
                num = lax.dot_general(
                    w0, v0, (((1,), (1,)), ((), ())),
                    preferred_element_type=jnp.float32,
                ) + lax.dot_general(
                    w1, v1, (((1,), (1,)), ((), ())),
                    preferred_element_type=jnp.float32,
                )
                ctx = num / (jnp.sum(w0, axis=1, keepdims=True)
                             + jnp.sum(w1, axis=1, keepdims=True))
            ctx_scr[b, qb * BLK:(qb + 1) * BLK, h * DH:(h + 1) * DH] = ctx

        rdma_a = {}

        def out_chunks(qb):
            for b in range(B):
                rows = pl.ds(qb * BLK, BLK)
                o = lax.dot_general(
                    ctx_scr[b, qb * BLK:(qb + 1) * BLK, :], wo_v[...],
                    (((1,), (0,)), ((), ())),
                    preferred_element_type=jnp.float32,
                )
                out_ref[b, rows] = o
                sendbuf[0, b, rows] = o.astype(jnp.bfloat16)
                r = ar_rdma(0, b, qb, p_a)
                r.start()
                rdma_a[(b, qb)] = r

        relay_forward(0)
        wait_kv(0)
        for b in range(B):
            for h in (0, 1):
                attention(b, h, 0)
                attention(b, h, 1)
        relay_forward(1)
        wait_kv(1)
        for b in range(B):
            for h in (2, 3):
                attention(b, h, 0)
                attention(b, h, 1)
        stage_in[2].wait()
        out_chunks(0)
        out_chunks(1)

        @pl.when(my == 0)
        def _():
            for r in scatter_sends():
                r.wait_send()

        for dev, kv in ((1, 0), (3, 1)):
            @pl.when(my == dev)
            def _():
                for c in range(2):
                    fwd_rdma(kv, c).wait_send()

        chunks = [(0, 0), (1, 0), (0, 1), (1, 1)]
        rdma_b = {}
        for b, qb in chunks:
            rows = pl.ds(qb * BLK, BLK)
            rdma_a[(b, qb)].wait()
            t = (out_ref[b, rows]
                 + comm[0, b, qb * BLK:(qb + 1) * BLK].astype(jnp.float32))
            out_ref[b, rows] = t
            sendbuf[1, b, rows] = t.astype(jnp.bfloat16)
            r = ar_rdma(1, b, qb, p_b)
            r.start()
            rdma_b[(b, qb)] = r
        for b, qb in chunks:
            rows = pl.ds(qb * BLK, BLK)
            rdma_b[(b, qb)].wait()
            out_ref[b, rows] = (out_ref[b, rows]
                                + comm[1, b, qb * BLK:(qb + 1) * BLK].astype(jnp.float32))

    return pl.pallas_call(
        body,
        out_shape=jax.ShapeDtypeStruct((B, SQ, D_MODEL), jnp.float32),
        in_specs=[pl.BlockSpec(memory_space=pl.ANY)] * 5,
        out_specs=pl.BlockSpec(memory_space=pltpu.VMEM),
        scratch_shapes=[
            pltpu.VMEM((B, SQ, D_MODEL), jnp.float32),
            pltpu.VMEM((D_MODEL, H_LOC * DH), jnp.float32),
            pltpu.VMEM((H_LOC * DH, D_MODEL), jnp.float32),
            pltpu.VMEM((3, 2, B, 2, DH, SQ), jnp.float32),
            pltpu.VMEM((B, SQ, H_LOC * DH), jnp.float32),
            pltpu.VMEM((B, SQ, H_LOC * DH), jnp.float32),
            pltpu.VMEM((2, B, SQ, D_MODEL), jnp.bfloat16),
            pltpu.VMEM((2, B, SQ, D_MODEL), jnp.bfloat16),
            pltpu.SemaphoreType.DMA((3,)),
            pltpu.SemaphoreType.DMA((12,)),
            pltpu.SemaphoreType.DMA((2, 2)),
            pltpu.SemaphoreType.DMA((2,)),
            pltpu.SemaphoreType.DMA((2,)),
            pltpu.SemaphoreType.DMA((2, 2)),
            pltpu.SemaphoreType.DMA((2, B, 2)),
            pltpu.SemaphoreType.DMA((2, B, 2)),
        ],
        compiler_params=pltpu.CompilerParams(collective_id=0),
    )(x, Wq, kT, vT, Wo)


# device time: 23189 ns/iter; 2.1605x vs baseline; 1.2777x over previous
import jax
import jax.numpy as jnp
from jax import lax
from jax.experimental import pallas as pl
from jax.experimental.pallas import tpu as pltpu

N_DEV = 4
B = 2
SQ = 128
H_LOC = 4
DH = 64
D_MODEL = 512
BLK = 64
SCALE = 0.125


def kernel(x, Wq, K_ext, V_ext, Wo):
    kT = lax.transpose(K_ext, (0, 2, 3, 1)).astype(jnp.bfloat16)
    vT = lax.transpose(V_ext, (0, 2, 3, 1)).astype(jnp.bfloat16)

    def body(x_ref, wq_ref, k_ref, v_ref, wo_ref, out_ref,
             x_v, wq_v, wo_v, kv_buf, q_scr, ctx_scr, comm, sendbuf,
             in_sems, send_sems, kv_recv_sems, relay_recv_sems,
             fwd_send_sems, local_sems, ar_send_sems, ar_recv_sems):
        my = lax.axis_index("i")

        stage_in = [
            pltpu.make_async_copy(x_ref, x_v, in_sems.at[0]),
            pltpu.make_async_copy(wq_ref, wq_v, in_sems.at[1]),
            pltpu.make_async_copy(wo_ref, wo_v, in_sems.at[2]),
        ]
        for c in stage_in:
            c.start()

        barrier_sem = pltpu.get_barrier_semaphore()

        @pl.when(my != 0)
        def _():
            pl.semaphore_signal(
                barrier_sem, inc=1, device_id=(0,),
                device_id_type=pl.DeviceIdType.MESH,
            )

        @pl.when(my == 0)
        def _():
            pl.semaphore_wait(barrier_sem, N_DEV - 1)

        def src_slice(kv, c, t):
            ref = {0: k_ref, 1: v_ref}[kv]
            return ref.at[:, pl.ds(H_LOC * t + 2 * c, 2), :, :]

        def dst_slice(kv, c):
            return kv_buf.at[kv, c]

        def scatter_sends():
            plan = []
            for c in range(2):
                plan.append((src_slice(0, c, 2), kv_buf.at[2, c],
                             relay_recv_sems.at[c], 1))
                plan.append((src_slice(1, c, 2), kv_buf.at[2, c],
                             relay_recv_sems.at[c], 3))
                for t in (1, 3):
                    for kv in range(2):
                        plan.append((src_slice(kv, c, t), dst_slice(kv, c),
                                     kv_recv_sems.at[kv, c], t))
            return [
                pltpu.make_async_remote_copy(
                    src_ref=s, dst_ref=d, send_sem=send_sems.at[i],
                    recv_sem=r, device_id=(t,),
                    device_id_type=pl.DeviceIdType.MESH,
                )
                for i, (s, d, r, t) in enumerate(plan)
            ]

        def local_copy(kv, c):
            return pltpu.make_async_copy(
                src_slice(kv, c, 0), dst_slice(kv, c),
                local_sems.at[kv, c],
            )

        @pl.when(my == 0)
        def _():
            for r in scatter_sends():
                r.start()
            for kv in range(2):
                for c in range(2):
                    local_copy(kv, c).start()

        stage_in[0].wait()
        stage_in[1].wait()
        for b in range(B):
            q_scr[b] = lax.dot_general(
                x_v[b], wq_v[...],
                (((1,), (0,)), ((), ())),
                preferred_element_type=jnp.float32,
            ).astype(jnp.bfloat16)

        def fwd_rdma(kv, c):
            return pltpu.make_async_remote_copy(
                src_ref=kv_buf.at[2, c],
                dst_ref=dst_slice(kv, c),
                send_sem=fwd_send_sems.at[c],
                recv_sem=kv_recv_sems.at[kv, c],
                device_id=(2,),
                device_id_type=pl.DeviceIdType.MESH,
            )

        def relay_forward(c):
            for dev, kv in ((1, 0), (3, 1)):
                @pl.when(my == dev)
                def _():
                    pltpu.make_async_remote_copy(
                        src_ref=kv_buf.at[2, c],
                        dst_ref=kv_buf.at[2, c],
                        send_sem=send_sems.at[0],
                        recv_sem=relay_recv_sems.at[c],
                        device_id=(0,),
                        device_id_type=pl.DeviceIdType.MESH,
                    ).wait_recv()
                    fwd_rdma(kv, c).start()

        def wait_kv(c):
            @pl.when(my == 0)
            def _():
                for kv in range(2):
                    local_copy(kv, c).wait()

            @pl.when(my != 0)
            def _():
                for kv in range(2):
                    pltpu.make_async_remote_copy(
                        src_ref=dst_slice(kv, c),
                        dst_ref=dst_slice(kv, c),
                        send_sem=send_sems.at[0],
                        recv_sem=kv_recv_sems.at[kv, c],
                        device_id=(0,),
                        device_id_type=pl.DeviceIdType.MESH,
                    ).wait_recv()

        p_a = my ^ 1
        p_b = 3 - my

        def ar_rdma(stage, b, qb, partner):
            return pltpu.make_async_remote_copy(
                src_ref=sendbuf.at[stage, b, pl.ds(qb * BLK, BLK)],
                dst_ref=comm.at[stage, b, pl.ds(qb * BLK, BLK)],
                send_sem=ar_send_sems.at[stage, b, qb],
                recv_sem=ar_recv_sems.at[stage, b, qb],
                device_id=(partner,),
                device_id_type=pl.DeviceIdType.MESH,
            )

        def attention(b, h, qb):
            c, sub = h // 2, h % 2
            q = q_scr[b, qb * BLK:(qb + 1) * BLK, h * DH:(h + 1) * DH]
            k0 = kv_buf[0, c, b, sub, :, 0:BLK]
            v0 = kv_buf[1, c, b, sub, :, 0:BLK]
            s0 = lax.dot_general(
                q, k0, (((1,), (0,)), ((), ())),
                preferred_element_type=jnp.float32,
            ) * SCALE
            if qb == 0:
                m = jnp.max(s0, axis=1, keepdims=True)
                w = jnp.exp(s0 - m)
                ctx = lax.dot_general(
                    w.astype(jnp.bfloat16), v0, (((1,), (1,)), ((), ())),
                    preferred_element_type=jnp.float32,
                ) / jnp.sum(w, axis=1, keepdims=True)
            else:
                k1 = kv_buf[0, c, b, sub, :, BLK:SQ]
                v1 = kv_buf[1, c, b, sub, :, BLK:SQ]
                s1 = lax.dot_general(
                    q, k1, (((1,), (0,)), ((), ())),
                    preferred_element_type=jnp.float32,
                ) * SCALE
                m = jnp.maximum(jnp.max(s0, axis=1, keepdims=True),
                                jnp.max(s1, axis=1, keepdims=True))
                w0 = jnp.exp(s0 - m)
                w1 = jnp.exp(s1 - m)
                num = lax.dot_general(
                    w0.astype(jnp.bfloat16), v0, (((1,), (1,)), ((), ())),
                    preferred_element_type=jnp.float32,
                ) + lax.dot_general(
                    w1.astype(jnp.bfloat16), v1, (((1,), (1,)), ((), ())),
                    preferred_element_type=jnp.float32,
                )
                ctx = num / (jnp.sum(w0, axis=1, keepdims=True)
                             + jnp.sum(w1, axis=1, keepdims=True))
            ctx_scr[b, qb * BLK:(qb + 1) * BLK, h * DH:(h + 1) * DH] = ctx

        rdma_a = {}

        def out_chunks(qb):
            for b in range(B):
                rows = pl.ds(qb * BLK, BLK)
                o = lax.dot_general(
                    ctx_scr[b, qb * BLK:(qb + 1) * BLK, :], wo_v[...],
                    (((1,), (0,)), ((), ())),
                    preferred_element_type=jnp.float32,
                )
                out_ref[b, rows] = o
                sendbuf[0, b, rows] = o.astype(jnp.bfloat16)
                r = ar_rdma(0, b, qb, p_a)
                r.start()
                rdma_a[(b, qb)] = r

        relay_forward(0)
        wait_kv(0)
        for b in range(B):
            for h in (0, 1):
                attention(b, h, 0)
                attention(b, h, 1)
        relay_forward(1)
        wait_kv(1)
        for b in range(B):
            for h in (2, 3):
                attention(b, h, 0)
                attention(b, h, 1)
        stage_in[2].wait()
        out_chunks(0)
        out_chunks(1)

        @pl.when(my == 0)
        def _():
            for r in scatter_sends():
                r.wait_send()

        for dev, kv in ((1, 0), (3, 1)):
            @pl.when(my == dev)
            def _():
                for c in range(2):
                    fwd_rdma(kv, c).wait_send()

        chunks = [(0, 0), (1, 0), (0, 1), (1, 1)]
        rdma_b = {}
        for b, qb in chunks:
            rows = pl.ds(qb * BLK, BLK)
            rdma_a[(b, qb)].wait()
            t = (out_ref[b, rows]
                 + comm[0, b, qb * BLK:(qb + 1) * BLK].astype(jnp.float32))
            out_ref[b, rows] = t
            sendbuf[1, b, rows] = t.astype(jnp.bfloat16)
            r = ar_rdma(1, b, qb, p_b)
            r.start()
            rdma_b[(b, qb)] = r
        for b, qb in chunks:
            rows = pl.ds(qb * BLK, BLK)
            rdma_b[(b, qb)].wait()
            out_ref[b, rows] = (out_ref[b, rows]
                                + comm[1, b, qb * BLK:(qb + 1) * BLK].astype(jnp.float32))

    return pl.pallas_call(
        body,
        out_shape=jax.ShapeDtypeStruct((B, SQ, D_MODEL), jnp.float32),
        in_specs=[pl.BlockSpec(memory_space=pl.ANY)] * 5,
        out_specs=pl.BlockSpec(memory_space=pltpu.VMEM),
        scratch_shapes=[
            pltpu.VMEM((B, SQ, D_MODEL), jnp.float32),
            pltpu.VMEM((D_MODEL, H_LOC * DH), jnp.float32),
            pltpu.VMEM((H_LOC * DH, D_MODEL), jnp.float32),
            pltpu.VMEM((3, 2, B, 2, DH, SQ), jnp.bfloat16),
            pltpu.VMEM((B, SQ, H_LOC * DH), jnp.bfloat16),
            pltpu.VMEM((B, SQ, H_LOC * DH), jnp.float32),
            pltpu.VMEM((2, B, SQ, D_MODEL), jnp.bfloat16),
            pltpu.VMEM((2, B, SQ, D_MODEL), jnp.bfloat16),
            pltpu.SemaphoreType.DMA((3,)),
            pltpu.SemaphoreType.DMA((12,)),
            pltpu.SemaphoreType.DMA((2, 2)),
            pltpu.SemaphoreType.DMA((2,)),
            pltpu.SemaphoreType.DMA((2,)),
            pltpu.SemaphoreType.DMA((2, 2)),
            pltpu.SemaphoreType.DMA((2, B, 2)),
            pltpu.SemaphoreType.DMA((2, B, 2)),
        ],
        compiler_params=pltpu.CompilerParams(collective_id=0),
    )(x, Wq, kT, vT, Wo)
